# baseline (speedup 1.0000x reference)
import jax
import jax.numpy as jnp
from jax import lax
from jax.experimental import pallas as pl
from jax.experimental.pallas import tpu as pltpu
from jax.experimental.pallas import tpu_sc as plsc

N = 10000
D = 128
E = 320000

NC = 2
NS = 16
NQ = 4
QD = D // NQ

CHUNK = 128
CPE = 160
HCPE = CPE // NC
EP = NS * CPE * CHUNK
NPAD = 10240
RPS = NPAD // NS
TRS = N // NS
NSLOT = 4
K = 3

_mesh = plsc.VectorSubcoreMesh(
    core_axis_name="c", subcore_axis_name="s", num_cores=NC, num_subcores=NS
)


def _sc_deg_body(dst_hbm, degp_hbm, dstv, ones_v, zero_v, dacc, sem):
    del sem
    c = lax.axis_index("c")
    s = lax.axis_index("s")
    for i in range(CHUNK // 16):
        ones_v[pl.ds(i * 16, 16)] = jnp.ones((16,), jnp.float32)
        zero_v[pl.ds(i * 16, 16)] = jnp.zeros((16,), jnp.float32)
    for r in range(RPS // CHUNK):
        pltpu.sync_copy(zero_v, dacc.at[pl.ds(s * RPS + r * CHUNK, CHUNK)])
    plsc.subcore_barrier()
    pltpu.sync_copy(dst_hbm.at[s].at[pl.ds(c * HCPE, HCPE)], dstv)

    @pl.loop(0, HCPE)
    def _(j):
        pltpu.sync_copy(ones_v, dacc.at[dstv.at[j]], add=True)

    plsc.subcore_barrier()
    pltpu.sync_copy(dacc.at[pl.ds(s * RPS, RPS)], degp_hbm.at[c, pl.ds(s * RPS, RPS)])


@jax.jit
def _sc_deg(dst3):
    kern = pl.kernel(
        _sc_deg_body,
        out_type=jax.ShapeDtypeStruct((NC, NPAD), jnp.float32),
        mesh=_mesh,
        scratch_types=[
            pltpu.VMEM((HCPE, CHUNK), jnp.int32),
            pltpu.VMEM((CHUNK,), jnp.float32),
            pltpu.VMEM((CHUNK,), jnp.float32),
            pltpu.VMEM_SHARED((NPAD,), jnp.float32),
            pltpu.SemaphoreType.DMA,
        ],
    )
    return kern(dst3)


def _sc_agg_body(hs_hbm, src_hbm, dst_hbm, out_hbm, srcv, dstv, *rest):
    bufs = list(rest[:NSLOT])
    zblk = rest[NSLOT]
    tbl = rest[NSLOT + 1]
    acc = rest[NSLOT + 2]
    gsems = list(rest[NSLOT + 3:NSLOT + 3 + NSLOT])
    ssems = list(rest[NSLOT + 3 + NSLOT:])
    c = lax.axis_index("c")
    s = lax.axis_index("s")
    pltpu.sync_copy(src_hbm.at[s], srcv)
    pltpu.sync_copy(dst_hbm.at[s], dstv)
    for i in range(CHUNK):
        for j in range(QD // 16):
            zblk.at[i, pl.ds(j * 16, 16)][...] = jnp.zeros((16,), jnp.float32)

    for q in range(NC):
        qi = c * NC + q
        pltpu.async_copy(
            hs_hbm.at[pl.ds(s * TRS, TRS), pl.ds(qi * QD, QD)],
            tbl.at[pl.ds(s * TRS, TRS)],
            gsems[0],
        )
        for r in range(RPS // CHUNK):
            pltpu.async_copy(
                zblk, acc.at[pl.ds(s * RPS + r * CHUNK, CHUNK)], gsems[1]
            )
        pltpu.make_async_copy(
            hs_hbm.at[pl.ds(s * TRS, TRS), pl.ds(qi * QD, QD)],
            tbl.at[pl.ds(s * TRS, TRS)],
            gsems[0],
        ).wait()
        for r in range(RPS // CHUNK):
            pltpu.make_async_copy(
                zblk, acc.at[pl.ds(s * RPS + r * CHUNK, CHUNK)], gsems[1]
            ).wait()
        plsc.subcore_barrier()

        for b in range(K):
            pltpu.async_copy(tbl.at[srcv.at[b]], bufs[b], gsems[b])

        @pl.loop(0, CPE, step=NSLOT)
        def _(j0):
            for b in range(NSLOT):
                jj = j0 + b
                nxt = jj + K
                bn = (b + K) % NSLOT

                @pl.when(jnp.logical_and(nxt < CPE, nxt >= NSLOT))
                def _():
                    pltpu.make_async_copy(
                        bufs[bn], acc.at[dstv.at[nxt - NSLOT]], ssems[bn]
                    ).wait()

                @pl.when(nxt < CPE)
                def _():
                    pltpu.async_copy(tbl.at[srcv.at[nxt]], bufs[bn], gsems[bn])

                pltpu.make_async_copy(tbl.at[srcv.at[jj]], bufs[b], gsems[b]).wait()
                pltpu.async_copy(bufs[b], acc.at[dstv.at[jj]], ssems[b], add=True)

        for b in range(NSLOT):
            pltpu.make_async_copy(
                bufs[b], acc.at[dstv.at[CPE - NSLOT + b]], ssems[b]
            ).wait()

        plsc.subcore_barrier()
        pltpu.sync_copy(
            acc.at[pl.ds(s * RPS, RPS)],
            out_hbm.at[pl.ds(s * RPS, RPS), pl.ds(qi * QD, QD)],
        )
        plsc.subcore_barrier()


@jax.jit
def _sc_agg(hs, src3, dst3):
    kern = pl.kernel(
        _sc_agg_body,
        out_type=jax.ShapeDtypeStruct((NPAD, D), jnp.float32),
        mesh=_mesh,
        compiler_params=pltpu.CompilerParams(use_tc_tiling_on_sc=False),
        scratch_types=[
            pltpu.VMEM((CPE, CHUNK), jnp.int32),
            pltpu.VMEM((CPE, CHUNK), jnp.int32),
        ]
        + [pltpu.VMEM((CHUNK, QD), jnp.float32) for _ in range(NSLOT)]
        + [
            pltpu.VMEM((CHUNK, QD), jnp.float32),
            pltpu.VMEM_SHARED((N, QD), jnp.float32),
            pltpu.VMEM_SHARED((NPAD, QD), jnp.float32),
        ]
        + [pltpu.SemaphoreType.DMA for _ in range(2 * NSLOT)],
    )
    return kern(hs, src3, dst3)


def _tc1_body(degp_ref, x_ref, w_ref, dinv_ref, hs_ref):
    deg = degp_ref[0, :N] + degp_ref[1, :N] + 1.0
    di = lax.rsqrt(deg)[:, None]
    dinv_ref[...] = di
    hs_ref[...] = (
        jnp.dot(x_ref[...], w_ref[...], preferred_element_type=jnp.float32) * di
    )


def _tc2_body(accp_ref, hs_ref, dinv_ref, b_ref, w_ref, out_ref):
    di = dinv_ref[...]
    z = (accp_ref[:N, :] + hs_ref[...]) * di + b_ref[...]
    h = jnp.maximum(z, 0.0)
    out_ref[...] = (
        jnp.dot(h, w_ref[...], preferred_element_type=jnp.float32) * di
    )


def _tc3_body(accp_ref, hs_ref, dinv_ref, b_ref, out_ref):
    di = dinv_ref[...]
    out_ref[...] = (accp_ref[:N, :] + hs_ref[...]) * di + b_ref[...]


@jax.jit
def _run(x, src3, dst3, W1, b1, W2, b2):
    degp = _sc_deg(dst3)
    dinv, hs1 = pl.pallas_call(
        _tc1_body,
        out_shape=(
            jax.ShapeDtypeStruct((N, 1), jnp.float32),
            jax.ShapeDtypeStruct((N, D), jnp.float32),
        ),
    )(degp, x, W1)
    acc1 = _sc_agg(hs1, src3, dst3)
    hs2 = pl.pallas_call(
        _tc2_body,
        out_shape=jax.ShapeDtypeStruct((N, D), jnp.float32),
    )(acc1, hs1, dinv, b1, W2)
    acc2 = _sc_agg(hs2, src3, dst3)
    out = pl.pallas_call(
        _tc3_body,
        out_shape=jax.ShapeDtypeStruct((N, D), jnp.float32),
    )(acc2, hs2, dinv, b2)
    return out


def kernel(x, edge_index, W1, b1, W2, b2):
    src = edge_index[0].astype(jnp.int32)
    dst = edge_index[1].astype(jnp.int32)
    pad = EP - E
    src3 = jnp.concatenate([src, jnp.zeros((pad,), jnp.int32)]).reshape(NS, CPE, CHUNK)
    dst3 = jnp.concatenate([dst, jnp.full((pad,), N, jnp.int32)]).reshape(NS, CPE, CHUNK)
    return _run(x, src3, dst3, W1, b1, W2, b2)


# speedup vs baseline: 1.1003x; 1.1003x over previous
import jax
import jax.numpy as jnp
from jax import lax
from jax.experimental import pallas as pl
from jax.experimental.pallas import tpu as pltpu
from jax.experimental.pallas import tpu_sc as plsc

N = 10000
D = 128
E = 320000

NC = 2
NS = 16
NQ = 4
QD = D // NQ

CHUNK = 128
CPE = 160
HCPE = CPE // NC
EP = NS * CPE * CHUNK
NPAD = 10240
RPS = NPAD // NS
TRS = N // NS
NSLOT = 4
K = 2

_mesh = plsc.VectorSubcoreMesh(
    core_axis_name="c", subcore_axis_name="s", num_cores=NC, num_subcores=NS
)


def _sc_deg_body(dst_hbm, degp_hbm, dstv, ones_v, zero_v, dacc, sem):
    del sem
    c = lax.axis_index("c")
    s = lax.axis_index("s")
    for i in range(CHUNK // 16):
        ones_v[pl.ds(i * 16, 16)] = jnp.ones((16,), jnp.float32)
        zero_v[pl.ds(i * 16, 16)] = jnp.zeros((16,), jnp.float32)
    for r in range(RPS // CHUNK):
        pltpu.sync_copy(zero_v, dacc.at[pl.ds(s * RPS + r * CHUNK, CHUNK)])
    plsc.subcore_barrier()
    pltpu.sync_copy(dst_hbm.at[s].at[pl.ds(c * HCPE, HCPE)], dstv)

    @pl.loop(0, HCPE)
    def _(j):
        pltpu.sync_copy(ones_v, dacc.at[dstv.at[j]], add=True)

    plsc.subcore_barrier()
    pltpu.sync_copy(dacc.at[pl.ds(s * RPS, RPS)], degp_hbm.at[c, pl.ds(s * RPS, RPS)])


@jax.jit
def _sc_deg(dst3):
    kern = pl.kernel(
        _sc_deg_body,
        out_type=jax.ShapeDtypeStruct((NC, NPAD), jnp.float32),
        mesh=_mesh,
        scratch_types=[
            pltpu.VMEM((HCPE, CHUNK), jnp.int32),
            pltpu.VMEM((CHUNK,), jnp.float32),
            pltpu.VMEM((CHUNK,), jnp.float32),
            pltpu.VMEM_SHARED((NPAD,), jnp.float32),
            pltpu.SemaphoreType.DMA,
        ],
    )
    return kern(dst3)


def _sc_agg_body(hs_hbm, src_hbm, dst_hbm, out_hbm, srcv, dstv, *rest):
    bufs = list(rest[:NSLOT])
    zblk = rest[NSLOT]
    tbl = rest[NSLOT + 1]
    acc = rest[NSLOT + 2]
    gsems = list(rest[NSLOT + 3:NSLOT + 3 + NSLOT])
    ssems = list(rest[NSLOT + 3 + NSLOT:])
    c = lax.axis_index("c")
    s = lax.axis_index("s")
    pltpu.sync_copy(src_hbm.at[s], srcv)
    pltpu.sync_copy(dst_hbm.at[s], dstv)
    for i in range(CHUNK):
        for j in range(QD // 16):
            zblk.at[i, pl.ds(j * 16, 16)][...] = jnp.zeros((16,), jnp.float32)

    for q in range(NC):
        qi = c * NC + q
        pltpu.async_copy(
            hs_hbm.at[pl.ds(s * TRS, TRS), pl.ds(qi * QD, QD)],
            tbl.at[pl.ds(s * TRS, TRS)],
            gsems[0],
        )
        for r in range(RPS // CHUNK):
            pltpu.async_copy(
                zblk, acc.at[pl.ds(s * RPS + r * CHUNK, CHUNK)], gsems[1]
            )
        pltpu.make_async_copy(
            hs_hbm.at[pl.ds(s * TRS, TRS), pl.ds(qi * QD, QD)],
            tbl.at[pl.ds(s * TRS, TRS)],
            gsems[0],
        ).wait()
        for r in range(RPS // CHUNK):
            pltpu.make_async_copy(
                zblk, acc.at[pl.ds(s * RPS + r * CHUNK, CHUNK)], gsems[1]
            ).wait()
        plsc.subcore_barrier()

        for b in range(K):
            pltpu.async_copy(tbl.at[srcv.at[b]], bufs[b], gsems[b])

        @pl.loop(0, CPE, step=NSLOT)
        def _(j0):
            for b in range(NSLOT):
                jj = j0 + b
                nxt = jj + K
                bn = (b + K) % NSLOT

                @pl.when(jnp.logical_and(nxt < CPE, nxt >= NSLOT))
                def _():
                    pltpu.make_async_copy(
                        bufs[bn], acc.at[dstv.at[nxt - NSLOT]], ssems[bn]
                    ).wait()

                @pl.when(nxt < CPE)
                def _():
                    pltpu.async_copy(tbl.at[srcv.at[nxt]], bufs[bn], gsems[bn])

                pltpu.make_async_copy(tbl.at[srcv.at[jj]], bufs[b], gsems[b]).wait()
                pltpu.async_copy(bufs[b], acc.at[dstv.at[jj]], ssems[b], add=True)

        for b in range(NSLOT):
            pltpu.make_async_copy(
                bufs[b], acc.at[dstv.at[CPE - NSLOT + b]], ssems[b]
            ).wait()

        plsc.subcore_barrier()
        pltpu.sync_copy(
            acc.at[pl.ds(s * RPS, RPS)],
            out_hbm.at[pl.ds(s * RPS, RPS), pl.ds(qi * QD, QD)],
        )
        plsc.subcore_barrier()


@jax.jit
def _sc_agg(hs, src3, dst3):
    kern = pl.kernel(
        _sc_agg_body,
        out_type=jax.ShapeDtypeStruct((NPAD, D), jnp.float32),
        mesh=_mesh,
        compiler_params=pltpu.CompilerParams(use_tc_tiling_on_sc=False),
        scratch_types=[
            pltpu.VMEM((CPE, CHUNK), jnp.int32),
            pltpu.VMEM((CPE, CHUNK), jnp.int32),
        ]
        + [pltpu.VMEM((CHUNK, QD), jnp.float32) for _ in range(NSLOT)]
        + [
            pltpu.VMEM((CHUNK, QD), jnp.float32),
            pltpu.VMEM_SHARED((N, QD), jnp.float32),
            pltpu.VMEM_SHARED((NPAD, QD), jnp.float32),
        ]
        + [pltpu.SemaphoreType.DMA for _ in range(2 * NSLOT)],
    )
    return kern(hs, src3, dst3)


def _tc1_body(degp_ref, x_ref, w_ref, dinv_ref, hs_ref):
    deg = degp_ref[0, :N] + degp_ref[1, :N] + 1.0
    di = lax.rsqrt(deg)[:, None]
    dinv_ref[...] = di
    hs_ref[...] = (
        jnp.dot(x_ref[...], w_ref[...], preferred_element_type=jnp.float32) * di
    )


def _tc2_body(accp_ref, hs_ref, dinv_ref, b_ref, w_ref, out_ref):
    di = dinv_ref[...]
    z = (accp_ref[:N, :] + hs_ref[...]) * di + b_ref[...]
    h = jnp.maximum(z, 0.0)
    out_ref[...] = (
        jnp.dot(h, w_ref[...], preferred_element_type=jnp.float32) * di
    )


def _tc3_body(accp_ref, hs_ref, dinv_ref, b_ref, out_ref):
    di = dinv_ref[...]
    out_ref[...] = (accp_ref[:N, :] + hs_ref[...]) * di + b_ref[...]


@jax.jit
def _run(x, src3, dst3, W1, b1, W2, b2):
    degp = _sc_deg(dst3)
    dinv, hs1 = pl.pallas_call(
        _tc1_body,
        out_shape=(
            jax.ShapeDtypeStruct((N, 1), jnp.float32),
            jax.ShapeDtypeStruct((N, D), jnp.float32),
        ),
    )(degp, x, W1)
    acc1 = _sc_agg(hs1, src3, dst3)
    hs2 = pl.pallas_call(
        _tc2_body,
        out_shape=jax.ShapeDtypeStruct((N, D), jnp.float32),
    )(acc1, hs1, dinv, b1, W2)
    acc2 = _sc_agg(hs2, src3, dst3)
    out = pl.pallas_call(
        _tc3_body,
        out_shape=jax.ShapeDtypeStruct((N, D), jnp.float32),
    )(acc2, hs2, dinv, b2)
    return out


def kernel(x, edge_index, W1, b1, W2, b2):
    src = edge_index[0].astype(jnp.int32)
    dst = edge_index[1].astype(jnp.int32)
    pad = EP - E
    src3 = jnp.concatenate([src, jnp.zeros((pad,), jnp.int32)]).reshape(NS, CPE, CHUNK)
    dst3 = jnp.concatenate([dst, jnp.full((pad,), N, jnp.int32)]).reshape(NS, CPE, CHUNK)
    return _run(x, src3, dst3, W1, b1, W2, b2)
